# baseline (speedup 1.0000x reference)
import functools

import jax
import jax.numpy as jnp
from jax import lax
from jax.experimental import pallas as pl
from jax.experimental.pallas import tpu as pltpu
from jax.experimental.pallas import tpu_sc as plsc

NC = 2
NS = 16
CH = 80

_NEG_INF = -3.0e38


def _lane_gather(vec, idx):
  dn = lax.GatherDimensionNumbers(
      offset_dims=(), collapsed_slice_dims=(0,), start_index_map=(0,))
  return lax.gather(vec, idx[:, None], dn, slice_sizes=(1,),
                    mode=lax.GatherScatterMode.PROMISE_IN_BOUNDS)


def _lane_bcast(vec, h):
  return _lane_gather(vec, jnp.full((16,), h, dtype=jnp.int32))


def _phase_a_body(x_ref, w1_ref, as_ref, ad_ref,
                  tab_ref, ald_ref, gmax_ref):
  b = x_ref.shape[0]
  h = jnp.dot(x_ref[...], w1_ref[...], preferred_element_type=jnp.float32)
  h3 = h.reshape(b, 8, 16)
  als = (h3 * as_ref[...][None]).sum(-1)
  ald = (h3 * ad_ref[...][None]).sum(-1)
  zeros8 = jnp.zeros((b, 8), jnp.float32)
  tab_ref[:, 0:128] = h
  tab_ref[:, 128:136] = jnp.ones((b, 8), jnp.float32)
  tab_ref[:, 136:144] = zeros8
  tab_ref[:, 144:152] = als
  tab_ref[:, 152:160] = zeros8
  ald_ref[:, 0:8] = ald
  ald_ref[:, 8:16] = zeros8
  bm = jnp.concatenate(
      [als.max(axis=0, keepdims=True), ald.max(axis=0, keepdims=True)], axis=1)

  @pl.when(pl.program_id(0) == 0)
  def _():
    gmax_ref[...] = bm

  @pl.when(pl.program_id(0) != 0)
  def _():
    gmax_ref[...] = jnp.maximum(gmax_ref[...], bm)


def _phase_a(x, w1, a_src1, a_dst1, n, bn):
  grid = (n // bn,)
  return pl.pallas_call(
      _phase_a_body,
      grid=grid,
      in_specs=[
          pl.BlockSpec((bn, 128), lambda i: (i, 0)),
          pl.BlockSpec((128, 128), lambda i: (0, 0)),
          pl.BlockSpec((8, 16), lambda i: (0, 0)),
          pl.BlockSpec((8, 16), lambda i: (0, 0)),
      ],
      out_specs=[
          pl.BlockSpec((bn, 160), lambda i: (i, 0)),
          pl.BlockSpec((bn, 16), lambda i: (i, 0)),
          pl.BlockSpec((1, 16), lambda i: (0, 0)),
      ],
      out_shape=[
          jax.ShapeDtypeStruct((n, 160), jnp.float32),
          jax.ShapeDtypeStruct((n, 16), jnp.float32),
          jax.ShapeDtypeStruct((1, 16), jnp.float32),
      ],
  )(x, w1, a_src1, a_dst1)


NSLOT = 2
CSUP = 32


def _make_edge_kernel(n_nodes, n_edges, epw, ch, sw, n_heads):
  gw = sw + 16
  n_chunks = epw // ch
  n_sup = n_chunks // CSUP
  npairs = CSUP // NSLOT
  rpt = (n_nodes // NS) // 8 * 8
  rpt_last = n_nodes - (NS - 1) * rpt
  mesh = plsc.VectorSubcoreMesh(
      core_axis_name="c", subcore_axis_name="s",
      num_cores=NC, num_subcores=NS)

  @functools.partial(
      pl.kernel,
      out_type=jax.ShapeDtypeStruct((NC * n_nodes, sw), jnp.float32),
      mesh=mesh,
      compiler_params=pltpu.CompilerParams(use_tc_tiling_on_sc=False),
      scratch_types=[
          pltpu.VMEM_SHARED((n_nodes, sw), jnp.float32),
          [pltpu.VMEM((CSUP, ch), jnp.int32)] * 2,
          [pltpu.VMEM((CSUP, ch), jnp.int32)] * 2,
          [pltpu.VMEM((ch, 16), jnp.float32)] * NSLOT,
          [pltpu.VMEM((ch, gw), jnp.float32)] * NSLOT,
          [pltpu.VMEM((ch, sw), jnp.float32)] * NSLOT,
          pltpu.VMEM((16,), jnp.float32),
          [pltpu.SemaphoreType.DMA] * NSLOT,
          [pltpu.SemaphoreType.DMA] * NSLOT,
      ],
  )
  def kern(src_hbm, dst_hbm, tab_hbm, ald_hbm, gmax_hbm, zeros_hbm,
           out_hbm, acc, sidx, didx, arows_d, rows, srows, gvec,
           gsem, ssem):
    c = lax.axis_index("c")
    s = lax.axis_index("s")
    wid = c * NS + s
    base = wid * epw

    def real(ci):
      return base + ci * ch < n_edges

    def issue_gathers(j, b, m):
      pltpu.async_copy(ald_hbm.at[didx[b].at[m]], arows_d[j], gsem[j])
      pltpu.async_copy(tab_hbm.at[sidx[b].at[m]], rows[j], gsem[j])

    def drain_gathers(j):
      pltpu.make_async_copy(
          ald_hbm.at[pl.ds(0, ch)], arows_d[j], gsem[j]).wait()
      pltpu.make_async_copy(
          tab_hbm.at[pl.ds(0, ch)], rows[j], gsem[j]).wait()

    def drain_scatter(j):
      pltpu.make_async_copy(
          zeros_hbm.at[pl.ds(0, ch)], srows[j], ssem[j]).wait()

    def fetch_idx(b, sp):
      off = sp * CSUP
      pltpu.sync_copy(src_hbm.at[wid, pl.ds(off, CSUP)], sidx[b])
      pltpu.sync_copy(dst_hbm.at[wid, pl.ds(off, CSUP)], didx[b])

    r0 = pl.multiple_of(s * rpt, 8)

    @pl.when(s != NS - 1)
    def _():
      pltpu.sync_copy(zeros_hbm.at[pl.ds(r0, rpt)], acc.at[pl.ds(r0, rpt)])

    @pl.when(s == NS - 1)
    def _():
      rl = (NS - 1) * rpt
      pltpu.sync_copy(zeros_hbm.at[pl.ds(rl, rpt_last)],
                      acc.at[pl.ds(rl, rpt_last)])

    pltpu.sync_copy(gmax_hbm.at[0], gvec)
    gv = gvec[...]
    rot = jnp.minimum(lax.iota(jnp.int32, 16) + 8, 15)
    g = jnp.maximum(gv + _lane_gather(gv, rot), 0.0)

    plsc.subcore_barrier()

    def compute(j):
      ard, rws, srws = arows_d[j], rows[j], srows[j]

      @pl.loop(0, ch, unroll=2)
      def _edge(e):
        raw = rws[e, pl.ds(sw, 16)] + ard[e, :]
        lr = jnp.maximum(raw, 0.2 * raw)
        w = jnp.exp(lr - g)
        for h in range(n_heads):
          wrep = _lane_bcast(w, h)
          srws[e, pl.ds(h * 16, 16)] = rws[e, pl.ds(h * 16, 16)] * wrep
        srws[e, pl.ds(n_heads * 16, 16)] = rws[e, pl.ds(n_heads * 16, 16)] * w

    def process(j, b, m, ci):
      drain_gathers(j)

      @pl.when((ci >= NSLOT) & real(ci - NSLOT))
      def _():
        drain_scatter(j)

      @pl.when(real(ci))
      def _():
        compute(j)
        pltpu.async_copy(srows[j], acc.at[didx[b].at[m]], ssem[j], add=True)

    fetch_idx(0, 0)
    for j in range(NSLOT):
      issue_gathers(j, 0, j)

    @pl.loop(0, n_sup // 2)
    def _sup2(t):
      for parity in range(2):
        sp = t * 2 + parity
        b, bn = parity, 1 - parity
        cbase = sp * CSUP

        for j in range(NSLOT):
          process(j, b, j, cbase + j)
          issue_gathers(j, b, j + NSLOT)

        @pl.when(sp + 1 < n_sup)
        def _():
          fetch_idx(bn, sp + 1)

        @pl.loop(1, npairs - 1)
        def _pair(k):
          for j in range(NSLOT):
            m = k * NSLOT + j
            process(j, b, m, cbase + m)
            issue_gathers(j, b, m + NSLOT)

        for j in range(NSLOT):
          m = (npairs - 1) * NSLOT + j
          process(j, b, m, cbase + m)

          @pl.when(sp + 1 < n_sup)
          def _():
            issue_gathers(j, bn, j)

    for j in range(NSLOT):
      ci = n_chunks - NSLOT + j

      @pl.when(real(ci))
      def _():
        drain_scatter(j)

    plsc.subcore_barrier()

    @pl.when(s != NS - 1)
    def _():
      pltpu.sync_copy(acc.at[pl.ds(r0, rpt)],
                      out_hbm.at[pl.ds(pl.multiple_of(c * n_nodes + r0, 8),
                                       rpt)])

    @pl.when(s == NS - 1)
    def _():
      rl = (NS - 1) * rpt
      pltpu.sync_copy(acc.at[pl.ds(rl, rpt_last)],
                      out_hbm.at[pl.ds(pl.multiple_of(c * n_nodes + rl, 8),
                                       rpt_last)])

  return kern


def _phase_b_body(p0_ref, p1_ref, tab1_ref, ald_ref, gmax_ref,
                  b1_ref, w2_ref, as2_ref, ad2_ref,
                  tab2_ref, ald2_ref, gmax2_ref):
  b = p0_ref.shape[0]
  acc = p0_ref[...] + p1_ref[...]
  a_s = tab1_ref[:, 144:152]
  a_d = ald_ref[:, 0:8]
  gm = gmax_ref[...]
  g = jnp.maximum(gm[:, 0:8] + gm[:, 8:16], 0.0)
  raw = a_s + a_d
  lr = jnp.maximum(raw, 0.2 * raw)
  wself = jnp.exp(lr - g)
  h1 = tab1_ref[:, 0:128]
  wrep = jnp.broadcast_to(wself[:, :, None], (b, 8, 16)).reshape(b, 128)
  num = acc[:, 0:128] + wrep * h1
  den = acc[:, 128:136] + wself
  den_rep = jnp.broadcast_to(den[:, :, None], (b, 8, 16)).reshape(b, 128)
  o1 = num / (den_rep + 1e-16) + b1_ref[...]
  x2 = jnp.where(o1 > 0, o1, jnp.exp(jnp.minimum(o1, 0.0)) - 1.0)
  h2 = jnp.dot(x2, w2_ref[...], preferred_element_type=jnp.float32)
  as2 = (h2 * as2_ref[...]).sum(axis=1, keepdims=True)
  ad2 = (h2 * ad2_ref[...]).sum(axis=1, keepdims=True)
  z15 = jnp.zeros((b, 15), jnp.float32)
  tab2_ref[:, 0:16] = h2
  tab2_ref[:, 16:17] = jnp.ones((b, 1), jnp.float32)
  tab2_ref[:, 17:32] = jnp.zeros((b, 15), jnp.float32)
  tab2_ref[:, 32:33] = as2
  tab2_ref[:, 33:48] = z15
  ald2_ref[:, 0:1] = ad2
  ald2_ref[:, 1:16] = z15
  cols = lax.broadcasted_iota(jnp.int32, (1, 16), 1)
  bm = jnp.where(cols == 0, as2.max(axis=0, keepdims=True),
                 jnp.where(cols == 8, ad2.max(axis=0, keepdims=True),
                           _NEG_INF))

  @pl.when(pl.program_id(0) == 0)
  def _():
    gmax2_ref[...] = bm

  @pl.when(pl.program_id(0) != 0)
  def _():
    gmax2_ref[...] = jnp.maximum(gmax2_ref[...], bm)


def _phase_b(p, tab1, ald1, gmax1, b1, w2, a_src2, a_dst2, n, bn):
  grid = (n // bn,)
  nb = n // bn
  return pl.pallas_call(
      _phase_b_body,
      grid=grid,
      in_specs=[
          pl.BlockSpec((bn, 144), lambda i: (i, 0)),
          pl.BlockSpec((bn, 144), lambda i, nb=nb: (i + nb, 0)),
          pl.BlockSpec((bn, 160), lambda i: (i, 0)),
          pl.BlockSpec((bn, 16), lambda i: (i, 0)),
          pl.BlockSpec((1, 16), lambda i: (0, 0)),
          pl.BlockSpec((1, 128), lambda i: (0, 0)),
          pl.BlockSpec((128, 16), lambda i: (0, 0)),
          pl.BlockSpec((1, 16), lambda i: (0, 0)),
          pl.BlockSpec((1, 16), lambda i: (0, 0)),
      ],
      out_specs=[
          pl.BlockSpec((bn, 48), lambda i: (i, 0)),
          pl.BlockSpec((bn, 16), lambda i: (i, 0)),
          pl.BlockSpec((1, 16), lambda i: (0, 0)),
      ],
      out_shape=[
          jax.ShapeDtypeStruct((n, 48), jnp.float32),
          jax.ShapeDtypeStruct((n, 16), jnp.float32),
          jax.ShapeDtypeStruct((1, 16), jnp.float32),
      ],
  )(p, p, tab1, ald1, gmax1, b1, w2, a_src2, a_dst2)


def _phase_c_body(q0_ref, q1_ref, tab2_ref, ald2_ref, gmax2_ref,
                  b2_ref, out_ref):
  acc = q0_ref[...] + q1_ref[...]
  a_s = tab2_ref[:, 32:33]
  a_d = ald2_ref[:, 0:1]
  gm = gmax2_ref[...]
  g = jnp.maximum(gm[:, 0:1] + gm[:, 8:9], 0.0)
  raw = a_s + a_d
  lr = jnp.maximum(raw, 0.2 * raw)
  wself = jnp.exp(lr - g)
  h2 = tab2_ref[:, 0:16]
  num = acc[:, 0:16] + wself * h2
  den = acc[:, 16:17] + wself
  out_ref[...] = num / (den + 1e-16) + b2_ref[...]


def _phase_c(q, tab2, ald2, gmax2, b2, n, bn):
  grid = (n // bn,)
  nb = n // bn
  return pl.pallas_call(
      _phase_c_body,
      grid=grid,
      in_specs=[
          pl.BlockSpec((bn, 32), lambda i: (i, 0)),
          pl.BlockSpec((bn, 32), lambda i, nb=nb: (i + nb, 0)),
          pl.BlockSpec((bn, 48), lambda i: (i, 0)),
          pl.BlockSpec((bn, 16), lambda i: (i, 0)),
          pl.BlockSpec((1, 16), lambda i: (0, 0)),
          pl.BlockSpec((1, 16), lambda i: (0, 0)),
      ],
      out_specs=pl.BlockSpec((bn, 16), lambda i: (i, 0)),
      out_shape=jax.ShapeDtypeStruct((n, 16), jnp.float32),
  )(q, q, tab2, ald2, gmax2, b2)


def kernel(x, edge_index, W1, a_src1, a_dst1, b1, W2, a_src2, a_dst2, b2):
  n = x.shape[0]
  e = edge_index.shape[1]
  bn = 1000
  nw = NC * NS
  ch1, ch2 = 40, 80
  grain = ch2 * CSUP * 2
  epw = (e + nw - 1) // nw
  epw = (epw + grain - 1) // grain * grain
  pad = nw * epw - e
  src = edge_index[0].astype(jnp.int32)
  dst = edge_index[1].astype(jnp.int32)
  if pad:
    zpad = jnp.zeros((pad,), jnp.int32)
    src = jnp.concatenate([src, zpad])
    dst = jnp.concatenate([dst, zpad])
  src1 = src.reshape(nw, epw // ch1, ch1)
  dst1 = dst.reshape(nw, epw // ch1, ch1)
  src2 = src.reshape(nw, epw // ch2, ch2)
  dst2 = dst.reshape(nw, epw // ch2, ch2)

  tab1, ald1, gmax1 = _phase_a(x, W1, a_src1, a_dst1, n, bn)

  zeros1 = jnp.zeros((n, 144), jnp.float32)
  sc1 = _make_edge_kernel(n, e, epw, ch1, 144, 8)
  p = sc1(src1, dst1, tab1, ald1, gmax1, zeros1)

  tab2, ald2, gmax2 = _phase_b(
      p, tab1, ald1, gmax1, b1.reshape(1, 128), W2, a_src2, a_dst2, n, bn)

  zeros2 = jnp.zeros((n, 32), jnp.float32)
  sc2 = _make_edge_kernel(n, e, epw, ch2, 32, 1)
  q = sc2(src2, dst2, tab2, ald2, gmax2, zeros2)

  return _phase_c(q, tab2, ald2, gmax2, b2.reshape(1, 16), n, bn)


# speedup vs baseline: 1.0773x; 1.0773x over previous
import functools
from math import gcd as _gcd

import jax
import jax.numpy as jnp
from jax import lax
from jax.experimental import pallas as pl
from jax.experimental.pallas import tpu as pltpu
from jax.experimental.pallas import tpu_sc as plsc

NC = 2
NS = 16
CH = 80

_NEG_INF = -3.0e38


def _lane_gather(vec, idx):
  dn = lax.GatherDimensionNumbers(
      offset_dims=(), collapsed_slice_dims=(0,), start_index_map=(0,))
  return lax.gather(vec, idx[:, None], dn, slice_sizes=(1,),
                    mode=lax.GatherScatterMode.PROMISE_IN_BOUNDS)


def _lane_bcast(vec, h):
  return _lane_gather(vec, jnp.full((16,), h, dtype=jnp.int32))


def _phase_a_body(x_ref, w1_ref, as_ref, ad_ref,
                  tab_ref, als_ref, ald_ref, gmax_ref):
  b = x_ref.shape[0]
  h = jnp.dot(x_ref[...], w1_ref[...], preferred_element_type=jnp.float32)
  h3 = h.reshape(b, 8, 16)
  als = (h3 * as_ref[...][None]).sum(-1)
  ald = (h3 * ad_ref[...][None]).sum(-1)
  zeros8 = jnp.zeros((b, 8), jnp.float32)
  tab_ref[:, 0:128] = h
  tab_ref[:, 128:136] = jnp.ones((b, 8), jnp.float32)
  tab_ref[:, 136:144] = zeros8
  als_ref[:, 0:8] = als
  als_ref[:, 8:16] = zeros8
  ald_ref[:, 0:8] = ald
  ald_ref[:, 8:16] = zeros8
  bm = jnp.concatenate(
      [als.max(axis=0, keepdims=True), ald.max(axis=0, keepdims=True)], axis=1)

  @pl.when(pl.program_id(0) == 0)
  def _():
    gmax_ref[...] = bm

  @pl.when(pl.program_id(0) != 0)
  def _():
    gmax_ref[...] = jnp.maximum(gmax_ref[...], bm)


def _phase_a(x, w1, a_src1, a_dst1, n, bn):
  grid = (n // bn,)
  return pl.pallas_call(
      _phase_a_body,
      grid=grid,
      in_specs=[
          pl.BlockSpec((bn, 128), lambda i: (i, 0)),
          pl.BlockSpec((128, 128), lambda i: (0, 0)),
          pl.BlockSpec((8, 16), lambda i: (0, 0)),
          pl.BlockSpec((8, 16), lambda i: (0, 0)),
      ],
      out_specs=[
          pl.BlockSpec((bn, 144), lambda i: (i, 0)),
          pl.BlockSpec((bn, 16), lambda i: (i, 0)),
          pl.BlockSpec((bn, 16), lambda i: (i, 0)),
          pl.BlockSpec((1, 16), lambda i: (0, 0)),
      ],
      out_shape=[
          jax.ShapeDtypeStruct((n, 144), jnp.float32),
          jax.ShapeDtypeStruct((n, 16), jnp.float32),
          jax.ShapeDtypeStruct((n, 16), jnp.float32),
          jax.ShapeDtypeStruct((1, 16), jnp.float32),
      ],
  )(x, w1, a_src1, a_dst1)


def _make_edge_kernel(n_nodes, n_edges, epw, ch, sw, n_heads, fused, nslot,
                      csup):
  gw = sw + 16 if fused else sw
  n_chunks = epw // ch
  n_sup = n_chunks // csup
  npairs = csup // nslot
  rpt = (n_nodes // NS) // 8 * 8
  rpt_last = n_nodes - (NS - 1) * rpt
  mesh = plsc.VectorSubcoreMesh(
      core_axis_name="c", subcore_axis_name="s",
      num_cores=NC, num_subcores=NS)

  scratch = [
      pltpu.VMEM_SHARED((n_nodes, sw), jnp.float32),
      [pltpu.VMEM((csup, ch), jnp.int32)] * 2,
      [pltpu.VMEM((csup, ch), jnp.int32)] * 2,
      [pltpu.VMEM((ch, 16), jnp.float32)] * nslot,
      [pltpu.VMEM((ch, gw), jnp.float32)] * nslot,
      [pltpu.VMEM((ch, sw), jnp.float32)] * nslot if fused else None,
      None if fused else [pltpu.VMEM((ch, 16), jnp.float32)] * nslot,
      pltpu.VMEM((16,), jnp.float32),
      [pltpu.SemaphoreType.DMA] * nslot,
      [pltpu.SemaphoreType.DMA] * nslot,
  ]
  scratch = [x for x in scratch if x is not None]

  @functools.partial(
      pl.kernel,
      out_type=jax.ShapeDtypeStruct((NC * n_nodes, sw), jnp.float32),
      mesh=mesh,
      compiler_params=pltpu.CompilerParams(use_tc_tiling_on_sc=False),
      scratch_types=scratch,
  )
  def kern(src_hbm, dst_hbm, tab_hbm, *rest):
    if fused:
      (ald_hbm, gmax_hbm, zeros_hbm, out_hbm, acc, sidx, didx, arows_d,
       rows, srows, gvec, gsem, ssem) = rest
      arows_s = None
    else:
      (als_hbm, ald_hbm, gmax_hbm, zeros_hbm, out_hbm, acc, sidx, didx,
       arows_d, rows, arows_s, gvec, gsem, ssem) = rest
      srows = rows
    c = lax.axis_index("c")
    s = lax.axis_index("s")
    wid = c * NS + s
    base = wid * epw

    def real(ci):
      return base + ci * ch < n_edges

    def issue_gathers(j, b, m):
      pltpu.async_copy(ald_hbm.at[didx[b].at[m]], arows_d[j], gsem[j])
      pltpu.async_copy(tab_hbm.at[sidx[b].at[m]], rows[j], gsem[j])
      if not fused:
        pltpu.async_copy(als_hbm.at[sidx[b].at[m]], arows_s[j], gsem[j])

    def drain_gathers(j):
      pltpu.make_async_copy(
          ald_hbm.at[pl.ds(0, ch)], arows_d[j], gsem[j]).wait()
      pltpu.make_async_copy(
          tab_hbm.at[pl.ds(0, ch)], rows[j], gsem[j]).wait()
      if not fused:
        pltpu.make_async_copy(
            ald_hbm.at[pl.ds(0, ch)], arows_s[j], gsem[j]).wait()

    def drain_scatter(j):
      pltpu.make_async_copy(
          zeros_hbm.at[pl.ds(0, ch)], srows[j], ssem[j]).wait()

    def fetch_idx(b, sp):
      off = sp * csup
      pltpu.sync_copy(src_hbm.at[wid, pl.ds(off, csup)], sidx[b])
      pltpu.sync_copy(dst_hbm.at[wid, pl.ds(off, csup)], didx[b])

    r0 = pl.multiple_of(s * rpt, 8)

    @pl.when(s != NS - 1)
    def _():
      pltpu.sync_copy(zeros_hbm.at[pl.ds(r0, rpt)], acc.at[pl.ds(r0, rpt)])

    @pl.when(s == NS - 1)
    def _():
      rl = (NS - 1) * rpt
      pltpu.sync_copy(zeros_hbm.at[pl.ds(rl, rpt_last)],
                      acc.at[pl.ds(rl, rpt_last)])

    pltpu.sync_copy(gmax_hbm.at[0], gvec)
    gv = gvec[...]
    rot = jnp.minimum(lax.iota(jnp.int32, 16) + 8, 15)
    g = jnp.maximum(gv + _lane_gather(gv, rot), 0.0)

    plsc.subcore_barrier()

    def compute(j):
      ard, rws, srws = arows_d[j], rows[j], srows[j]
      ars = None if fused else arows_s[j]

      @pl.loop(0, ch, unroll=2)
      def _edge(e):
        a_s = rws[e, pl.ds(sw, 16)] if fused else ars[e, :]
        raw = a_s + ard[e, :]
        lr = jnp.maximum(raw, 0.2 * raw)
        w = jnp.exp(lr - g)
        for h in range(n_heads):
          wrep = _lane_bcast(w, h)
          srws[e, pl.ds(h * 16, 16)] = rws[e, pl.ds(h * 16, 16)] * wrep
        srws[e, pl.ds(n_heads * 16, 16)] = rws[e, pl.ds(n_heads * 16, 16)] * w

    def issue_scatter(j, b, m):
      pltpu.async_copy(srows[j], acc.at[didx[b].at[m]], ssem[j], add=True)

    def process(j, b, m, ci):
      drain_gathers(j)

      if fused:
        @pl.when((ci >= nslot) & real(ci - nslot))
        def _():
          drain_scatter(j)

      @pl.when(real(ci))
      def _():
        compute(j)
        issue_scatter(j, b, m)

    fetch_idx(0, 0)
    for j in range(nslot):
      issue_gathers(j, 0, j)

    @pl.loop(0, n_sup // 2)
    def _sup2(t):
      for parity in range(2):
        sp = t * 2 + parity
        b, bn = parity, 1 - parity
        cbase = sp * csup

        if fused:
          for j in range(nslot):
            process(j, b, j, cbase + j)
            issue_gathers(j, b, j + nslot)

          @pl.when(sp + 1 < n_sup)
          def _():
            fetch_idx(bn, sp + 1)

          @pl.loop(1, npairs - 1)
          def _grp(k):
            for j in range(nslot):
              m = k * nslot + j
              process(j, b, m, cbase + m)
              issue_gathers(j, b, m + nslot)

          for j in range(nslot):
            m = (npairs - 1) * nslot + j
            process(j, b, m, cbase + m)

            @pl.when(sp + 1 < n_sup)
            def _():
              issue_gathers(j, bn, j)
        else:
          @pl.when(sp + 1 < n_sup)
          def _():
            fetch_idx(bn, sp + 1)

          @pl.loop(0, npairs - 1)
          def _grp(k):
            for j in range(nslot):
              m = k * nslot + j
              process(j, b, m, cbase + m)
            for j in range(nslot):
              m = k * nslot + j

              @pl.when(real(cbase + m))
              def _():
                drain_scatter(j)

              issue_gathers(j, b, m + nslot)

          for j in range(nslot):
            m = (npairs - 1) * nslot + j
            process(j, b, m, cbase + m)

          for j in range(nslot):
            m = (npairs - 1) * nslot + j

            @pl.when(real(cbase + m))
            def _():
              drain_scatter(j)

            @pl.when(sp + 1 < n_sup)
            def _():
              issue_gathers(j, bn, j)

    if fused:
      for j in range(nslot):
        ci = n_chunks - nslot + j

        @pl.when(real(ci))
        def _():
          drain_scatter(j)

    plsc.subcore_barrier()

    @pl.when(s != NS - 1)
    def _():
      pltpu.sync_copy(acc.at[pl.ds(r0, rpt)],
                      out_hbm.at[pl.ds(pl.multiple_of(c * n_nodes + r0, 8),
                                       rpt)])

    @pl.when(s == NS - 1)
    def _():
      rl = (NS - 1) * rpt
      pltpu.sync_copy(acc.at[pl.ds(rl, rpt_last)],
                      out_hbm.at[pl.ds(pl.multiple_of(c * n_nodes + rl, 8),
                                       rpt_last)])

  return kern


def _phase_b_body(p0_ref, p1_ref, tab1_ref, als_ref, ald_ref, gmax_ref,
                  b1_ref, w2_ref, as2_ref, ad2_ref,
                  tab2_ref, ald2_ref, gmax2_ref):
  b = p0_ref.shape[0]
  acc = p0_ref[...] + p1_ref[...]
  a_s = als_ref[:, 0:8]
  a_d = ald_ref[:, 0:8]
  gm = gmax_ref[...]
  g = jnp.maximum(gm[:, 0:8] + gm[:, 8:16], 0.0)
  raw = a_s + a_d
  lr = jnp.maximum(raw, 0.2 * raw)
  wself = jnp.exp(lr - g)
  h1 = tab1_ref[:, 0:128]
  wrep = jnp.broadcast_to(wself[:, :, None], (b, 8, 16)).reshape(b, 128)
  num = acc[:, 0:128] + wrep * h1
  den = acc[:, 128:136] + wself
  den_rep = jnp.broadcast_to(den[:, :, None], (b, 8, 16)).reshape(b, 128)
  o1 = num / (den_rep + 1e-16) + b1_ref[...]
  x2 = jnp.where(o1 > 0, o1, jnp.exp(jnp.minimum(o1, 0.0)) - 1.0)
  h2 = jnp.dot(x2, w2_ref[...], preferred_element_type=jnp.float32)
  as2 = (h2 * as2_ref[...]).sum(axis=1, keepdims=True)
  ad2 = (h2 * ad2_ref[...]).sum(axis=1, keepdims=True)
  z15 = jnp.zeros((b, 15), jnp.float32)
  tab2_ref[:, 0:16] = h2
  tab2_ref[:, 16:17] = jnp.ones((b, 1), jnp.float32)
  tab2_ref[:, 17:32] = jnp.zeros((b, 15), jnp.float32)
  tab2_ref[:, 32:33] = as2
  tab2_ref[:, 33:48] = z15
  ald2_ref[:, 0:1] = ad2
  ald2_ref[:, 1:16] = z15
  cols = lax.broadcasted_iota(jnp.int32, (1, 16), 1)
  bm = jnp.where(cols == 0, as2.max(axis=0, keepdims=True),
                 jnp.where(cols == 8, ad2.max(axis=0, keepdims=True),
                           _NEG_INF))

  @pl.when(pl.program_id(0) == 0)
  def _():
    gmax2_ref[...] = bm

  @pl.when(pl.program_id(0) != 0)
  def _():
    gmax2_ref[...] = jnp.maximum(gmax2_ref[...], bm)


def _phase_b(p, tab1, als1, ald1, gmax1, b1, w2, a_src2, a_dst2, n, bn):
  grid = (n // bn,)
  nb = n // bn
  return pl.pallas_call(
      _phase_b_body,
      grid=grid,
      in_specs=[
          pl.BlockSpec((bn, 144), lambda i: (i, 0)),
          pl.BlockSpec((bn, 144), lambda i, nb=nb: (i + nb, 0)),
          pl.BlockSpec((bn, 144), lambda i: (i, 0)),
          pl.BlockSpec((bn, 16), lambda i: (i, 0)),
          pl.BlockSpec((bn, 16), lambda i: (i, 0)),
          pl.BlockSpec((1, 16), lambda i: (0, 0)),
          pl.BlockSpec((1, 128), lambda i: (0, 0)),
          pl.BlockSpec((128, 16), lambda i: (0, 0)),
          pl.BlockSpec((1, 16), lambda i: (0, 0)),
          pl.BlockSpec((1, 16), lambda i: (0, 0)),
      ],
      out_specs=[
          pl.BlockSpec((bn, 48), lambda i: (i, 0)),
          pl.BlockSpec((bn, 16), lambda i: (i, 0)),
          pl.BlockSpec((1, 16), lambda i: (0, 0)),
      ],
      out_shape=[
          jax.ShapeDtypeStruct((n, 48), jnp.float32),
          jax.ShapeDtypeStruct((n, 16), jnp.float32),
          jax.ShapeDtypeStruct((1, 16), jnp.float32),
      ],
  )(p, p, tab1, als1, ald1, gmax1, b1, w2, a_src2, a_dst2)


def _phase_c_body(q0_ref, q1_ref, tab2_ref, ald2_ref, gmax2_ref,
                  b2_ref, out_ref):
  acc = q0_ref[...] + q1_ref[...]
  a_s = tab2_ref[:, 32:33]
  a_d = ald2_ref[:, 0:1]
  gm = gmax2_ref[...]
  g = jnp.maximum(gm[:, 0:1] + gm[:, 8:9], 0.0)
  raw = a_s + a_d
  lr = jnp.maximum(raw, 0.2 * raw)
  wself = jnp.exp(lr - g)
  h2 = tab2_ref[:, 0:16]
  num = acc[:, 0:16] + wself * h2
  den = acc[:, 16:17] + wself
  out_ref[...] = num / (den + 1e-16) + b2_ref[...]


def _phase_c(q, tab2, ald2, gmax2, b2, n, bn):
  grid = (n // bn,)
  nb = n // bn
  return pl.pallas_call(
      _phase_c_body,
      grid=grid,
      in_specs=[
          pl.BlockSpec((bn, 32), lambda i: (i, 0)),
          pl.BlockSpec((bn, 32), lambda i, nb=nb: (i + nb, 0)),
          pl.BlockSpec((bn, 48), lambda i: (i, 0)),
          pl.BlockSpec((bn, 16), lambda i: (i, 0)),
          pl.BlockSpec((1, 16), lambda i: (0, 0)),
          pl.BlockSpec((1, 16), lambda i: (0, 0)),
      ],
      out_specs=pl.BlockSpec((bn, 16), lambda i: (i, 0)),
      out_shape=jax.ShapeDtypeStruct((n, 16), jnp.float32),
  )(q, q, tab2, ald2, gmax2, b2)


def kernel(x, edge_index, W1, a_src1, a_dst1, b1, W2, a_src2, a_dst2, b2):
  n = x.shape[0]
  e = edge_index.shape[1]
  bn = 1000
  nw = NC * NS
  ch1, csup1 = 80, 16
  ch2, csup2 = 128, 8
  g1, g2 = ch1 * csup1 * 2, ch2 * csup2 * 2
  grain = g1 * g2 // _gcd(g1, g2)
  epw = (e + nw - 1) // nw
  epw = (epw + grain - 1) // grain * grain
  pad = nw * epw - e
  src = edge_index[0].astype(jnp.int32)
  dst = edge_index[1].astype(jnp.int32)
  if pad:
    zpad = jnp.zeros((pad,), jnp.int32)
    src = jnp.concatenate([src, zpad])
    dst = jnp.concatenate([dst, zpad])
  src1 = src.reshape(nw, epw // ch1, ch1)
  dst1 = dst.reshape(nw, epw // ch1, ch1)
  src2 = src.reshape(nw, epw // ch2, ch2)
  dst2 = dst.reshape(nw, epw // ch2, ch2)

  tab1, als1, ald1, gmax1 = _phase_a(x, W1, a_src1, a_dst1, n, bn)

  zeros1 = jnp.zeros((n, 144), jnp.float32)
  sc1 = _make_edge_kernel(n, e, epw, ch1, 144, 8, fused=False, nslot=2,
                          csup=csup1)
  p = sc1(src1, dst1, tab1, als1, ald1, gmax1, zeros1)

  tab2, ald2, gmax2 = _phase_b(
      p, tab1, als1, ald1, gmax1, b1.reshape(1, 128), W2, a_src2, a_dst2,
      n, bn)

  zeros2 = jnp.zeros((n, 32), jnp.float32)
  sc2 = _make_edge_kernel(n, e, epw, ch2, 32, 1, fused=True, nslot=4,
                          csup=csup2)
  q = sc2(src2, dst2, tab2, ald2, gmax2, zeros2)

  return _phase_c(q, tab2, ald2, gmax2, b2.reshape(1, 16), n, bn)


# speedup vs baseline: 1.2028x; 1.1164x over previous
import functools
from math import gcd as _gcd

import jax
import jax.numpy as jnp
from jax import lax
from jax.experimental import pallas as pl
from jax.experimental.pallas import tpu as pltpu
from jax.experimental.pallas import tpu_sc as plsc

NC = 2
NS = 16
CH = 80

_NEG_INF = -3.0e38


def _lane_gather(vec, idx):
  dn = lax.GatherDimensionNumbers(
      offset_dims=(), collapsed_slice_dims=(0,), start_index_map=(0,))
  return lax.gather(vec, idx[:, None], dn, slice_sizes=(1,),
                    mode=lax.GatherScatterMode.PROMISE_IN_BOUNDS)


def _lane_bcast(vec, h):
  return _lane_gather(vec, jnp.full((16,), h, dtype=jnp.int32))


def _phase_a_body(x_ref, w1_ref, as_ref, ad_ref,
                  tab_ref, als_ref, ald_ref, gmax_ref):
  b = x_ref.shape[0]
  h = jnp.dot(x_ref[...], w1_ref[...], preferred_element_type=jnp.float32)
  h3 = h.reshape(b, 8, 16)
  als = (h3 * as_ref[...][None]).sum(-1)
  ald = (h3 * ad_ref[...][None]).sum(-1)
  zeros8 = jnp.zeros((b, 8), jnp.float32)
  tab_ref[:, 0:128] = h
  tab_ref[:, 128:136] = jnp.ones((b, 8), jnp.float32)
  tab_ref[:, 136:144] = zeros8
  als_ref[:, 0:8] = als
  als_ref[:, 8:16] = zeros8
  ald_ref[:, 0:8] = ald
  ald_ref[:, 8:16] = zeros8
  bm = jnp.concatenate(
      [als.max(axis=0, keepdims=True), ald.max(axis=0, keepdims=True)], axis=1)

  @pl.when(pl.program_id(0) == 0)
  def _():
    gmax_ref[...] = bm

  @pl.when(pl.program_id(0) != 0)
  def _():
    gmax_ref[...] = jnp.maximum(gmax_ref[...], bm)


def _phase_a(x, w1, a_src1, a_dst1, n, bn):
  grid = (n // bn,)
  return pl.pallas_call(
      _phase_a_body,
      grid=grid,
      in_specs=[
          pl.BlockSpec((bn, 128), lambda i: (i, 0)),
          pl.BlockSpec((128, 128), lambda i: (0, 0)),
          pl.BlockSpec((8, 16), lambda i: (0, 0)),
          pl.BlockSpec((8, 16), lambda i: (0, 0)),
      ],
      out_specs=[
          pl.BlockSpec((bn, 144), lambda i: (i, 0)),
          pl.BlockSpec((bn, 16), lambda i: (i, 0)),
          pl.BlockSpec((bn, 16), lambda i: (i, 0)),
          pl.BlockSpec((1, 16), lambda i: (0, 0)),
      ],
      out_shape=[
          jax.ShapeDtypeStruct((n, 144), jnp.float32),
          jax.ShapeDtypeStruct((n, 16), jnp.float32),
          jax.ShapeDtypeStruct((n, 16), jnp.float32),
          jax.ShapeDtypeStruct((1, 16), jnp.float32),
      ],
  )(x, w1, a_src1, a_dst1)


def _make_edge_kernel(n_nodes, n_edges, epw, ch, sw, n_heads, fused, nslot,
                      csup):
  gw = sw + 16 if fused else sw
  n_chunks = epw // ch
  n_sup = n_chunks // csup
  npairs = csup // nslot
  rpt = (n_nodes // NS) // 8 * 8
  rpt_last = n_nodes - (NS - 1) * rpt
  mesh = plsc.VectorSubcoreMesh(
      core_axis_name="c", subcore_axis_name="s",
      num_cores=NC, num_subcores=NS)

  scratch = [
      pltpu.VMEM_SHARED((n_nodes, sw), jnp.float32),
      [pltpu.VMEM((csup, ch), jnp.int32)] * 2,
      [pltpu.VMEM((csup, ch), jnp.int32)] * 2,
      [pltpu.VMEM((ch, 16), jnp.float32)] * nslot,
      [pltpu.VMEM((ch, gw), jnp.float32)] * nslot,
      [pltpu.VMEM((ch, sw), jnp.float32)] * nslot if fused else None,
      None if fused else [pltpu.VMEM((ch, 16), jnp.float32)] * nslot,
      pltpu.VMEM((16,), jnp.float32),
      [pltpu.SemaphoreType.DMA] * nslot,
      [pltpu.SemaphoreType.DMA] * nslot,
  ]
  scratch = [x for x in scratch if x is not None]

  @functools.partial(
      pl.kernel,
      out_type=jax.ShapeDtypeStruct((NC * n_nodes, sw), jnp.float32),
      mesh=mesh,
      compiler_params=pltpu.CompilerParams(use_tc_tiling_on_sc=False),
      scratch_types=scratch,
  )
  def kern(src_hbm, dst_hbm, tab_hbm, *rest):
    if fused:
      (ald_hbm, gmax_hbm, zeros_hbm, out_hbm, acc, sidx, didx, arows_d,
       rows, srows, gvec, gsem, ssem) = rest
      arows_s = None
    else:
      (als_hbm, ald_hbm, gmax_hbm, zeros_hbm, out_hbm, acc, sidx, didx,
       arows_d, rows, arows_s, gvec, gsem, ssem) = rest
      srows = rows
    c = lax.axis_index("c")
    s = lax.axis_index("s")
    wid = c * NS + s
    base = wid * epw

    def real(ci):
      return base + ci * ch < n_edges

    def issue_gathers(j, b, m):
      pltpu.async_copy(ald_hbm.at[didx[b].at[m]], arows_d[j], gsem[j])
      pltpu.async_copy(tab_hbm.at[sidx[b].at[m]], rows[j], gsem[j])
      if not fused:
        pltpu.async_copy(als_hbm.at[sidx[b].at[m]], arows_s[j], gsem[j])

    def drain_gathers(j):
      pltpu.make_async_copy(
          ald_hbm.at[pl.ds(0, ch)], arows_d[j], gsem[j]).wait()
      pltpu.make_async_copy(
          tab_hbm.at[pl.ds(0, ch)], rows[j], gsem[j]).wait()
      if not fused:
        pltpu.make_async_copy(
            ald_hbm.at[pl.ds(0, ch)], arows_s[j], gsem[j]).wait()

    def drain_scatter(j):
      pltpu.make_async_copy(
          zeros_hbm.at[pl.ds(0, ch)], srows[j], ssem[j]).wait()

    def fetch_idx(b, sp):
      off = sp * csup
      pltpu.sync_copy(src_hbm.at[wid, pl.ds(off, csup)], sidx[b])
      pltpu.sync_copy(dst_hbm.at[wid, pl.ds(off, csup)], didx[b])

    r0 = pl.multiple_of(s * rpt, 8)

    @pl.when(s != NS - 1)
    def _():
      pltpu.sync_copy(zeros_hbm.at[pl.ds(r0, rpt)], acc.at[pl.ds(r0, rpt)])

    @pl.when(s == NS - 1)
    def _():
      rl = (NS - 1) * rpt
      pltpu.sync_copy(zeros_hbm.at[pl.ds(rl, rpt_last)],
                      acc.at[pl.ds(rl, rpt_last)])

    pltpu.sync_copy(gmax_hbm.at[0], gvec)
    gv = gvec[...]
    rot = jnp.minimum(lax.iota(jnp.int32, 16) + 8, 15)
    g = jnp.maximum(gv + _lane_gather(gv, rot), 0.0)

    plsc.subcore_barrier()

    def compute(j):
      ard, rws, srws = arows_d[j], rows[j], srows[j]
      ars = None if fused else arows_s[j]

      @plsc.parallel_loop(0, ch, unroll=4)
      def _edge(e):
        a_s = rws[e, pl.ds(sw, 16)] if fused else ars[e, :]
        raw = a_s + ard[e, :]
        lr = jnp.maximum(raw, 0.2 * raw)
        w = jnp.exp(lr - g)
        for h in range(n_heads):
          wrep = _lane_bcast(w, h)
          srws[e, pl.ds(h * 16, 16)] = rws[e, pl.ds(h * 16, 16)] * wrep
        srws[e, pl.ds(n_heads * 16, 16)] = rws[e, pl.ds(n_heads * 16, 16)] * w

    def issue_scatter(j, b, m):
      pltpu.async_copy(srows[j], acc.at[didx[b].at[m]], ssem[j], add=True)

    def process(j, b, m, ci):
      drain_gathers(j)

      if fused:
        @pl.when((ci >= nslot) & real(ci - nslot))
        def _():
          drain_scatter(j)

      @pl.when(real(ci))
      def _():
        compute(j)
        issue_scatter(j, b, m)

    fetch_idx(0, 0)
    for j in range(nslot):
      issue_gathers(j, 0, j)

    @pl.loop(0, n_sup // 2)
    def _sup2(t):
      for parity in range(2):
        sp = t * 2 + parity
        b, bn = parity, 1 - parity
        cbase = sp * csup

        if fused:
          for j in range(nslot):
            process(j, b, j, cbase + j)
            issue_gathers(j, b, j + nslot)

          @pl.when(sp + 1 < n_sup)
          def _():
            fetch_idx(bn, sp + 1)

          @pl.loop(1, npairs - 1)
          def _grp(k):
            for j in range(nslot):
              m = k * nslot + j
              process(j, b, m, cbase + m)
              issue_gathers(j, b, m + nslot)

          for j in range(nslot):
            m = (npairs - 1) * nslot + j
            process(j, b, m, cbase + m)

            @pl.when(sp + 1 < n_sup)
            def _():
              issue_gathers(j, bn, j)
        else:
          @pl.when(sp + 1 < n_sup)
          def _():
            fetch_idx(bn, sp + 1)

          @pl.loop(0, npairs - 1)
          def _grp(k):
            for j in range(nslot):
              m = k * nslot + j
              process(j, b, m, cbase + m)
            for j in range(nslot):
              m = k * nslot + j

              @pl.when(real(cbase + m))
              def _():
                drain_scatter(j)

              issue_gathers(j, b, m + nslot)

          for j in range(nslot):
            m = (npairs - 1) * nslot + j
            process(j, b, m, cbase + m)

          for j in range(nslot):
            m = (npairs - 1) * nslot + j

            @pl.when(real(cbase + m))
            def _():
              drain_scatter(j)

            @pl.when(sp + 1 < n_sup)
            def _():
              issue_gathers(j, bn, j)

    if fused:
      for j in range(nslot):
        ci = n_chunks - nslot + j

        @pl.when(real(ci))
        def _():
          drain_scatter(j)

    plsc.subcore_barrier()

    @pl.when(s != NS - 1)
    def _():
      pltpu.sync_copy(acc.at[pl.ds(r0, rpt)],
                      out_hbm.at[pl.ds(pl.multiple_of(c * n_nodes + r0, 8),
                                       rpt)])

    @pl.when(s == NS - 1)
    def _():
      rl = (NS - 1) * rpt
      pltpu.sync_copy(acc.at[pl.ds(rl, rpt_last)],
                      out_hbm.at[pl.ds(pl.multiple_of(c * n_nodes + rl, 8),
                                       rpt_last)])

  return kern


def _phase_b_body(p0_ref, p1_ref, tab1_ref, als_ref, ald_ref, gmax_ref,
                  b1_ref, w2_ref, as2_ref, ad2_ref,
                  tab2_ref, ald2_ref, gmax2_ref):
  b = p0_ref.shape[0]
  acc = p0_ref[...] + p1_ref[...]
  a_s = als_ref[:, 0:8]
  a_d = ald_ref[:, 0:8]
  gm = gmax_ref[...]
  g = jnp.maximum(gm[:, 0:8] + gm[:, 8:16], 0.0)
  raw = a_s + a_d
  lr = jnp.maximum(raw, 0.2 * raw)
  wself = jnp.exp(lr - g)
  h1 = tab1_ref[:, 0:128]
  wrep = jnp.broadcast_to(wself[:, :, None], (b, 8, 16)).reshape(b, 128)
  num = acc[:, 0:128] + wrep * h1
  den = acc[:, 128:136] + wself
  den_rep = jnp.broadcast_to(den[:, :, None], (b, 8, 16)).reshape(b, 128)
  o1 = num / (den_rep + 1e-16) + b1_ref[...]
  x2 = jnp.where(o1 > 0, o1, jnp.exp(jnp.minimum(o1, 0.0)) - 1.0)
  h2 = jnp.dot(x2, w2_ref[...], preferred_element_type=jnp.float32)
  as2 = (h2 * as2_ref[...]).sum(axis=1, keepdims=True)
  ad2 = (h2 * ad2_ref[...]).sum(axis=1, keepdims=True)
  z15 = jnp.zeros((b, 15), jnp.float32)
  tab2_ref[:, 0:16] = h2
  tab2_ref[:, 16:17] = jnp.ones((b, 1), jnp.float32)
  tab2_ref[:, 17:32] = jnp.zeros((b, 15), jnp.float32)
  tab2_ref[:, 32:33] = as2
  tab2_ref[:, 33:48] = z15
  ald2_ref[:, 0:1] = ad2
  ald2_ref[:, 1:16] = z15
  cols = lax.broadcasted_iota(jnp.int32, (1, 16), 1)
  bm = jnp.where(cols == 0, as2.max(axis=0, keepdims=True),
                 jnp.where(cols == 8, ad2.max(axis=0, keepdims=True),
                           _NEG_INF))

  @pl.when(pl.program_id(0) == 0)
  def _():
    gmax2_ref[...] = bm

  @pl.when(pl.program_id(0) != 0)
  def _():
    gmax2_ref[...] = jnp.maximum(gmax2_ref[...], bm)


def _phase_b(p, tab1, als1, ald1, gmax1, b1, w2, a_src2, a_dst2, n, bn):
  grid = (n // bn,)
  nb = n // bn
  return pl.pallas_call(
      _phase_b_body,
      grid=grid,
      in_specs=[
          pl.BlockSpec((bn, 144), lambda i: (i, 0)),
          pl.BlockSpec((bn, 144), lambda i, nb=nb: (i + nb, 0)),
          pl.BlockSpec((bn, 144), lambda i: (i, 0)),
          pl.BlockSpec((bn, 16), lambda i: (i, 0)),
          pl.BlockSpec((bn, 16), lambda i: (i, 0)),
          pl.BlockSpec((1, 16), lambda i: (0, 0)),
          pl.BlockSpec((1, 128), lambda i: (0, 0)),
          pl.BlockSpec((128, 16), lambda i: (0, 0)),
          pl.BlockSpec((1, 16), lambda i: (0, 0)),
          pl.BlockSpec((1, 16), lambda i: (0, 0)),
      ],
      out_specs=[
          pl.BlockSpec((bn, 48), lambda i: (i, 0)),
          pl.BlockSpec((bn, 16), lambda i: (i, 0)),
          pl.BlockSpec((1, 16), lambda i: (0, 0)),
      ],
      out_shape=[
          jax.ShapeDtypeStruct((n, 48), jnp.float32),
          jax.ShapeDtypeStruct((n, 16), jnp.float32),
          jax.ShapeDtypeStruct((1, 16), jnp.float32),
      ],
  )(p, p, tab1, als1, ald1, gmax1, b1, w2, a_src2, a_dst2)


def _phase_c_body(q0_ref, q1_ref, tab2_ref, ald2_ref, gmax2_ref,
                  b2_ref, out_ref):
  acc = q0_ref[...] + q1_ref[...]
  a_s = tab2_ref[:, 32:33]
  a_d = ald2_ref[:, 0:1]
  gm = gmax2_ref[...]
  g = jnp.maximum(gm[:, 0:1] + gm[:, 8:9], 0.0)
  raw = a_s + a_d
  lr = jnp.maximum(raw, 0.2 * raw)
  wself = jnp.exp(lr - g)
  h2 = tab2_ref[:, 0:16]
  num = acc[:, 0:16] + wself * h2
  den = acc[:, 16:17] + wself
  out_ref[...] = num / (den + 1e-16) + b2_ref[...]


def _phase_c(q, tab2, ald2, gmax2, b2, n, bn):
  grid = (n // bn,)
  nb = n // bn
  return pl.pallas_call(
      _phase_c_body,
      grid=grid,
      in_specs=[
          pl.BlockSpec((bn, 32), lambda i: (i, 0)),
          pl.BlockSpec((bn, 32), lambda i, nb=nb: (i + nb, 0)),
          pl.BlockSpec((bn, 48), lambda i: (i, 0)),
          pl.BlockSpec((bn, 16), lambda i: (i, 0)),
          pl.BlockSpec((1, 16), lambda i: (0, 0)),
          pl.BlockSpec((1, 16), lambda i: (0, 0)),
      ],
      out_specs=pl.BlockSpec((bn, 16), lambda i: (i, 0)),
      out_shape=jax.ShapeDtypeStruct((n, 16), jnp.float32),
  )(q, q, tab2, ald2, gmax2, b2)


def kernel(x, edge_index, W1, a_src1, a_dst1, b1, W2, a_src2, a_dst2, b2):
  n = x.shape[0]
  e = edge_index.shape[1]
  bn = 1000
  nw = NC * NS
  ch1, csup1 = 80, 16
  ch2, csup2 = 128, 8
  g1, g2 = ch1 * csup1 * 2, ch2 * csup2 * 2
  grain = g1 * g2 // _gcd(g1, g2)
  epw = (e + nw - 1) // nw
  epw = (epw + grain - 1) // grain * grain
  pad = nw * epw - e
  src = edge_index[0].astype(jnp.int32)
  dst = edge_index[1].astype(jnp.int32)
  if pad:
    zpad = jnp.zeros((pad,), jnp.int32)
    src = jnp.concatenate([src, zpad])
    dst = jnp.concatenate([dst, zpad])
  src1 = src.reshape(nw, epw // ch1, ch1)
  dst1 = dst.reshape(nw, epw // ch1, ch1)
  src2 = src.reshape(nw, epw // ch2, ch2)
  dst2 = dst.reshape(nw, epw // ch2, ch2)

  tab1, als1, ald1, gmax1 = _phase_a(x, W1, a_src1, a_dst1, n, bn)

  zeros1 = jnp.zeros((n, 144), jnp.float32)
  sc1 = _make_edge_kernel(n, e, epw, ch1, 144, 8, fused=False, nslot=2,
                          csup=csup1)
  p = sc1(src1, dst1, tab1, als1, ald1, gmax1, zeros1)

  tab2, ald2, gmax2 = _phase_b(
      p, tab1, als1, ald1, gmax1, b1.reshape(1, 128), W2, a_src2, a_dst2,
      n, bn)

  zeros2 = jnp.zeros((n, 32), jnp.float32)
  sc2 = _make_edge_kernel(n, e, epw, ch2, 32, 1, fused=True, nslot=4,
                          csup=csup2)
  q = sc2(src2, dst2, tab2, ald2, gmax2, zeros2)

  return _phase_c(q, tab2, ald2, gmax2, b2.reshape(1, 16), n, bn)


# speedup vs baseline: 1.2045x; 1.0014x over previous
import functools
from math import gcd as _gcd

import jax
import jax.numpy as jnp
from jax import lax
from jax.experimental import pallas as pl
from jax.experimental.pallas import tpu as pltpu
from jax.experimental.pallas import tpu_sc as plsc

NC = 2
NS = 16
CH = 80

_NEG_INF = -3.0e38


def _lane_gather(vec, idx):
  dn = lax.GatherDimensionNumbers(
      offset_dims=(), collapsed_slice_dims=(0,), start_index_map=(0,))
  return lax.gather(vec, idx[:, None], dn, slice_sizes=(1,),
                    mode=lax.GatherScatterMode.PROMISE_IN_BOUNDS)


def _lane_bcast(vec, h):
  return _lane_gather(vec, jnp.full((16,), h, dtype=jnp.int32))


def _phase_a_body(x_ref, w1_ref, as_ref, ad_ref,
                  tab_ref, als_ref, ald_ref, gmax_ref):
  b = x_ref.shape[0]
  h = jnp.dot(x_ref[...], w1_ref[...], preferred_element_type=jnp.float32)
  h3 = h.reshape(b, 8, 16)
  als = (h3 * as_ref[...][None]).sum(-1)
  ald = (h3 * ad_ref[...][None]).sum(-1)
  zeros8 = jnp.zeros((b, 8), jnp.float32)
  tab_ref[:, 0:128] = h
  tab_ref[:, 128:136] = jnp.ones((b, 8), jnp.float32)
  tab_ref[:, 136:144] = zeros8
  als_ref[:, 0:8] = als
  als_ref[:, 8:16] = zeros8
  ald_ref[:, 0:8] = ald
  ald_ref[:, 8:16] = zeros8
  bm = jnp.concatenate(
      [als.max(axis=0, keepdims=True), ald.max(axis=0, keepdims=True)], axis=1)

  @pl.when(pl.program_id(0) == 0)
  def _():
    gmax_ref[...] = bm

  @pl.when(pl.program_id(0) != 0)
  def _():
    gmax_ref[...] = jnp.maximum(gmax_ref[...], bm)


def _phase_a(x, w1, a_src1, a_dst1, n, bn):
  grid = (n // bn,)
  return pl.pallas_call(
      _phase_a_body,
      grid=grid,
      in_specs=[
          pl.BlockSpec((bn, 128), lambda i: (i, 0)),
          pl.BlockSpec((128, 128), lambda i: (0, 0)),
          pl.BlockSpec((8, 16), lambda i: (0, 0)),
          pl.BlockSpec((8, 16), lambda i: (0, 0)),
      ],
      out_specs=[
          pl.BlockSpec((bn, 144), lambda i: (i, 0)),
          pl.BlockSpec((bn, 16), lambda i: (i, 0)),
          pl.BlockSpec((bn, 16), lambda i: (i, 0)),
          pl.BlockSpec((1, 16), lambda i: (0, 0)),
      ],
      out_shape=[
          jax.ShapeDtypeStruct((n, 144), jnp.float32),
          jax.ShapeDtypeStruct((n, 16), jnp.float32),
          jax.ShapeDtypeStruct((n, 16), jnp.float32),
          jax.ShapeDtypeStruct((1, 16), jnp.float32),
      ],
  )(x, w1, a_src1, a_dst1)


def _make_edge_kernel(n_nodes, n_edges, epw, ch, sw, n_heads, fused, nslot,
                      csup):
  gw = sw + 16 if fused else sw
  n_chunks = epw // ch
  n_sup = n_chunks // csup
  npairs = csup // nslot
  rpt = (n_nodes // NS) // 8 * 8
  rpt_last = n_nodes - (NS - 1) * rpt
  mesh = plsc.VectorSubcoreMesh(
      core_axis_name="c", subcore_axis_name="s",
      num_cores=NC, num_subcores=NS)

  scratch = [
      pltpu.VMEM_SHARED((n_nodes, sw), jnp.float32),
      [pltpu.VMEM((csup, ch), jnp.int32)] * 2,
      [pltpu.VMEM((csup, ch), jnp.int32)] * 2,
      [pltpu.VMEM((ch, 16), jnp.float32)] * nslot,
      [pltpu.VMEM((ch, gw), jnp.float32)] * nslot,
      [pltpu.VMEM((ch, sw), jnp.float32)] * nslot if fused else None,
      None if fused else [pltpu.VMEM((ch, 16), jnp.float32)] * nslot,
      pltpu.VMEM((16,), jnp.float32),
      [pltpu.SemaphoreType.DMA] * nslot,
      [pltpu.SemaphoreType.DMA] * nslot,
  ]
  scratch = [x for x in scratch if x is not None]

  @functools.partial(
      pl.kernel,
      out_type=jax.ShapeDtypeStruct((NC * n_nodes, sw), jnp.float32),
      mesh=mesh,
      compiler_params=pltpu.CompilerParams(use_tc_tiling_on_sc=False),
      scratch_types=scratch,
  )
  def kern(src_hbm, dst_hbm, tab_hbm, *rest):
    if fused:
      (ald_hbm, gmax_hbm, zeros_hbm, out_hbm, acc, sidx, didx, arows_d,
       rows, srows, gvec, gsem, ssem) = rest
      arows_s = None
    else:
      (als_hbm, ald_hbm, gmax_hbm, zeros_hbm, out_hbm, acc, sidx, didx,
       arows_d, rows, arows_s, gvec, gsem, ssem) = rest
      srows = rows
    c = lax.axis_index("c")
    s = lax.axis_index("s")
    wid = c * NS + s
    base = wid * epw

    def real(ci):
      return base + ci * ch < n_edges

    def issue_gathers(j, b, m):
      pltpu.async_copy(ald_hbm.at[didx[b].at[m]], arows_d[j], gsem[j])
      pltpu.async_copy(tab_hbm.at[sidx[b].at[m]], rows[j], gsem[j])
      if not fused:
        pltpu.async_copy(als_hbm.at[sidx[b].at[m]], arows_s[j], gsem[j])

    def drain_gathers(j):
      pltpu.make_async_copy(
          ald_hbm.at[pl.ds(0, ch)], arows_d[j], gsem[j]).wait()
      pltpu.make_async_copy(
          tab_hbm.at[pl.ds(0, ch)], rows[j], gsem[j]).wait()
      if not fused:
        pltpu.make_async_copy(
            ald_hbm.at[pl.ds(0, ch)], arows_s[j], gsem[j]).wait()

    def drain_scatter(j):
      pltpu.make_async_copy(
          zeros_hbm.at[pl.ds(0, ch)], srows[j], ssem[j]).wait()

    def fetch_idx(b, sp):
      off = sp * csup
      pltpu.sync_copy(src_hbm.at[wid, pl.ds(off, csup)], sidx[b])
      pltpu.sync_copy(dst_hbm.at[wid, pl.ds(off, csup)], didx[b])

    r0 = pl.multiple_of(s * rpt, 8)

    @pl.when(s != NS - 1)
    def _():
      pltpu.sync_copy(zeros_hbm.at[pl.ds(r0, rpt)], acc.at[pl.ds(r0, rpt)])

    @pl.when(s == NS - 1)
    def _():
      rl = (NS - 1) * rpt
      pltpu.sync_copy(zeros_hbm.at[pl.ds(rl, rpt_last)],
                      acc.at[pl.ds(rl, rpt_last)])

    pltpu.sync_copy(gmax_hbm.at[0], gvec)
    gv = gvec[...]
    rot = jnp.minimum(lax.iota(jnp.int32, 16) + 8, 15)
    g = jnp.maximum(gv + _lane_gather(gv, rot), 0.0)

    plsc.subcore_barrier()

    def compute(j):
      ard, rws, srws = arows_d[j], rows[j], srows[j]
      ars = None if fused else arows_s[j]

      @plsc.parallel_loop(0, ch, unroll=4)
      def _edge(e):
        a_s = rws[e, pl.ds(sw, 16)] if fused else ars[e, :]
        raw = a_s + ard[e, :]
        lr = jnp.maximum(raw, 0.2 * raw)
        w = jnp.exp(lr - g)
        for h in range(n_heads):
          wrep = _lane_bcast(w, h)
          srws[e, pl.ds(h * 16, 16)] = rws[e, pl.ds(h * 16, 16)] * wrep
        srws[e, pl.ds(n_heads * 16, 16)] = rws[e, pl.ds(n_heads * 16, 16)] * w

    def issue_scatter(j, b, m):
      pltpu.async_copy(srows[j], acc.at[didx[b].at[m]], ssem[j], add=True)

    def process(j, b, m, ci):
      drain_gathers(j)

      if fused:
        @pl.when((ci >= nslot) & real(ci - nslot))
        def _():
          drain_scatter(j)

      @pl.when(real(ci))
      def _():
        compute(j)
        issue_scatter(j, b, m)

    fetch_idx(0, 0)
    for j in range(nslot):
      issue_gathers(j, 0, j)

    @pl.loop(0, n_sup // 2)
    def _sup2(t):
      for parity in range(2):
        sp = t * 2 + parity
        b, bn = parity, 1 - parity
        cbase = sp * csup

        if fused:
          for j in range(nslot):
            process(j, b, j, cbase + j)
            issue_gathers(j, b, j + nslot)

          @pl.when(sp + 1 < n_sup)
          def _():
            fetch_idx(bn, sp + 1)

          @pl.loop(1, npairs - 1)
          def _grp(k):
            for j in range(nslot):
              m = k * nslot + j
              process(j, b, m, cbase + m)
              issue_gathers(j, b, m + nslot)

          for j in range(nslot):
            m = (npairs - 1) * nslot + j
            process(j, b, m, cbase + m)

            @pl.when(sp + 1 < n_sup)
            def _():
              issue_gathers(j, bn, j)
        else:
          @pl.when(sp + 1 < n_sup)
          def _():
            fetch_idx(bn, sp + 1)

          @pl.loop(0, npairs - 1)
          def _grp(k):
            for j in range(nslot):
              m = k * nslot + j
              process(j, b, m, cbase + m)
            for j in range(nslot):
              m = k * nslot + j

              @pl.when(real(cbase + m))
              def _():
                drain_scatter(j)

              issue_gathers(j, b, m + nslot)

          for j in range(nslot):
            m = (npairs - 1) * nslot + j
            process(j, b, m, cbase + m)

          for j in range(nslot):
            m = (npairs - 1) * nslot + j

            @pl.when(real(cbase + m))
            def _():
              drain_scatter(j)

            @pl.when(sp + 1 < n_sup)
            def _():
              issue_gathers(j, bn, j)

    if fused:
      for j in range(nslot):
        ci = n_chunks - nslot + j

        @pl.when(real(ci))
        def _():
          drain_scatter(j)

    plsc.subcore_barrier()

    @pl.when(s != NS - 1)
    def _():
      pltpu.sync_copy(acc.at[pl.ds(r0, rpt)],
                      out_hbm.at[pl.ds(pl.multiple_of(c * n_nodes + r0, 8),
                                       rpt)])

    @pl.when(s == NS - 1)
    def _():
      rl = (NS - 1) * rpt
      pltpu.sync_copy(acc.at[pl.ds(rl, rpt_last)],
                      out_hbm.at[pl.ds(pl.multiple_of(c * n_nodes + rl, 8),
                                       rpt_last)])

  return kern


def _phase_b_body(p0_ref, p1_ref, tab1_ref, als_ref, ald_ref, gmax_ref,
                  b1_ref, w2_ref, as2_ref, ad2_ref,
                  tab2_ref, ald2_ref, gmax2_ref):
  b = p0_ref.shape[0]
  acc = p0_ref[...] + p1_ref[...]
  a_s = als_ref[:, 0:8]
  a_d = ald_ref[:, 0:8]
  gm = gmax_ref[...]
  g = jnp.maximum(gm[:, 0:8] + gm[:, 8:16], 0.0)
  raw = a_s + a_d
  lr = jnp.maximum(raw, 0.2 * raw)
  wself = jnp.exp(lr - g)
  h1 = tab1_ref[:, 0:128]
  wrep = jnp.broadcast_to(wself[:, :, None], (b, 8, 16)).reshape(b, 128)
  num = acc[:, 0:128] + wrep * h1
  den = acc[:, 128:136] + wself
  den_rep = jnp.broadcast_to(den[:, :, None], (b, 8, 16)).reshape(b, 128)
  o1 = num / (den_rep + 1e-16) + b1_ref[...]
  x2 = jnp.where(o1 > 0, o1, jnp.exp(jnp.minimum(o1, 0.0)) - 1.0)
  h2 = jnp.dot(x2, w2_ref[...], preferred_element_type=jnp.float32)
  as2 = (h2 * as2_ref[...]).sum(axis=1, keepdims=True)
  ad2 = (h2 * ad2_ref[...]).sum(axis=1, keepdims=True)
  z15 = jnp.zeros((b, 15), jnp.float32)
  tab2_ref[:, 0:16] = h2
  tab2_ref[:, 16:17] = jnp.ones((b, 1), jnp.float32)
  tab2_ref[:, 17:32] = jnp.zeros((b, 15), jnp.float32)
  tab2_ref[:, 32:33] = as2
  tab2_ref[:, 33:48] = z15
  ald2_ref[:, 0:1] = ad2
  ald2_ref[:, 1:16] = z15
  cols = lax.broadcasted_iota(jnp.int32, (1, 16), 1)
  bm = jnp.where(cols == 0, as2.max(axis=0, keepdims=True),
                 jnp.where(cols == 8, ad2.max(axis=0, keepdims=True),
                           _NEG_INF))

  @pl.when(pl.program_id(0) == 0)
  def _():
    gmax2_ref[...] = bm

  @pl.when(pl.program_id(0) != 0)
  def _():
    gmax2_ref[...] = jnp.maximum(gmax2_ref[...], bm)


def _phase_b(p, tab1, als1, ald1, gmax1, b1, w2, a_src2, a_dst2, n, bn):
  grid = (n // bn,)
  nb = n // bn
  return pl.pallas_call(
      _phase_b_body,
      grid=grid,
      in_specs=[
          pl.BlockSpec((bn, 144), lambda i: (i, 0)),
          pl.BlockSpec((bn, 144), lambda i, nb=nb: (i + nb, 0)),
          pl.BlockSpec((bn, 144), lambda i: (i, 0)),
          pl.BlockSpec((bn, 16), lambda i: (i, 0)),
          pl.BlockSpec((bn, 16), lambda i: (i, 0)),
          pl.BlockSpec((1, 16), lambda i: (0, 0)),
          pl.BlockSpec((1, 128), lambda i: (0, 0)),
          pl.BlockSpec((128, 16), lambda i: (0, 0)),
          pl.BlockSpec((1, 16), lambda i: (0, 0)),
          pl.BlockSpec((1, 16), lambda i: (0, 0)),
      ],
      out_specs=[
          pl.BlockSpec((bn, 48), lambda i: (i, 0)),
          pl.BlockSpec((bn, 16), lambda i: (i, 0)),
          pl.BlockSpec((1, 16), lambda i: (0, 0)),
      ],
      out_shape=[
          jax.ShapeDtypeStruct((n, 48), jnp.float32),
          jax.ShapeDtypeStruct((n, 16), jnp.float32),
          jax.ShapeDtypeStruct((1, 16), jnp.float32),
      ],
  )(p, p, tab1, als1, ald1, gmax1, b1, w2, a_src2, a_dst2)


def _phase_c_body(q0_ref, q1_ref, tab2_ref, ald2_ref, gmax2_ref,
                  b2_ref, out_ref):
  acc = q0_ref[...] + q1_ref[...]
  a_s = tab2_ref[:, 32:33]
  a_d = ald2_ref[:, 0:1]
  gm = gmax2_ref[...]
  g = jnp.maximum(gm[:, 0:1] + gm[:, 8:9], 0.0)
  raw = a_s + a_d
  lr = jnp.maximum(raw, 0.2 * raw)
  wself = jnp.exp(lr - g)
  h2 = tab2_ref[:, 0:16]
  num = acc[:, 0:16] + wself * h2
  den = acc[:, 16:17] + wself
  out_ref[...] = num / (den + 1e-16) + b2_ref[...]


def _phase_c(q, tab2, ald2, gmax2, b2, n, bn):
  grid = (n // bn,)
  nb = n // bn
  return pl.pallas_call(
      _phase_c_body,
      grid=grid,
      in_specs=[
          pl.BlockSpec((bn, 32), lambda i: (i, 0)),
          pl.BlockSpec((bn, 32), lambda i, nb=nb: (i + nb, 0)),
          pl.BlockSpec((bn, 48), lambda i: (i, 0)),
          pl.BlockSpec((bn, 16), lambda i: (i, 0)),
          pl.BlockSpec((1, 16), lambda i: (0, 0)),
          pl.BlockSpec((1, 16), lambda i: (0, 0)),
      ],
      out_specs=pl.BlockSpec((bn, 16), lambda i: (i, 0)),
      out_shape=jax.ShapeDtypeStruct((n, 16), jnp.float32),
  )(q, q, tab2, ald2, gmax2, b2)


def kernel(x, edge_index, W1, a_src1, a_dst1, b1, W2, a_src2, a_dst2, b2):
  n = x.shape[0]
  e = edge_index.shape[1]
  bn = 1000
  nw = NC * NS
  ch1, csup1 = 80, 32
  ch2, csup2 = 128, 8
  g1, g2 = ch1 * csup1 * 2, ch2 * csup2 * 2
  grain = g1 * g2 // _gcd(g1, g2)
  epw = (e + nw - 1) // nw
  epw = (epw + grain - 1) // grain * grain
  pad = nw * epw - e
  src = edge_index[0].astype(jnp.int32)
  dst = edge_index[1].astype(jnp.int32)
  if pad:
    zpad = jnp.zeros((pad,), jnp.int32)
    src = jnp.concatenate([src, zpad])
    dst = jnp.concatenate([dst, zpad])
  src1 = src.reshape(nw, epw // ch1, ch1)
  dst1 = dst.reshape(nw, epw // ch1, ch1)
  src2 = src.reshape(nw, epw // ch2, ch2)
  dst2 = dst.reshape(nw, epw // ch2, ch2)

  tab1, als1, ald1, gmax1 = _phase_a(x, W1, a_src1, a_dst1, n, bn)

  zeros1 = jnp.zeros((n, 144), jnp.float32)
  sc1 = _make_edge_kernel(n, e, epw, ch1, 144, 8, fused=False, nslot=2,
                          csup=csup1)
  p = sc1(src1, dst1, tab1, als1, ald1, gmax1, zeros1)

  tab2, ald2, gmax2 = _phase_b(
      p, tab1, als1, ald1, gmax1, b1.reshape(1, 128), W2, a_src2, a_dst2,
      n, bn)

  zeros2 = jnp.zeros((n, 32), jnp.float32)
  sc2 = _make_edge_kernel(n, e, epw, ch2, 32, 1, fused=True, nslot=4,
                          csup=csup2)
  q = sc2(src2, dst2, tab2, ald2, gmax2, zeros2)

  return _phase_c(q, tab2, ald2, gmax2, b2.reshape(1, 16), n, bn)
